# baseline (speedup 1.0000x reference)
import functools

import jax
import jax.numpy as jnp
from jax import lax
from jax.experimental import pallas as pl
from jax.experimental.pallas import tpu as pltpu
from jax.experimental.pallas import tpu_sc as plsc

L = 16
NC = 2
NS = 16
NW = NC * NS
B = 16384
D = 144
V = 180
VP = 192
ROWS = B // NW
BLK = 2048


_DOT_DIMS = (((1,), (0,)), ((), ()))


def _tc_s_body(pre_ref, w1_ref, s_ref):
    s = lax.dot_general(pre_ref[...], w1_ref[...], _DOT_DIMS,
                        preferred_element_type=jnp.float32)
    s_ref[...] = s[:, 0:1].reshape(VP)


def _tc_s(pre_pad, w1c):
    return pl.pallas_call(
        _tc_s_body,
        in_specs=[
            pl.BlockSpec((VP, D), lambda: (0, 0)),
            pl.BlockSpec((D, 128), lambda: (0, 0)),
        ],
        out_specs=pl.BlockSpec((VP,), lambda: (0,)),
        out_shape=jax.ShapeDtypeStruct((VP,), jnp.float32),
    )(pre_pad, w1c)


def _tc_y_body(emb_ref, w2_ref, y_ref):
    y = lax.dot_general(emb_ref[...], w2_ref[...], _DOT_DIMS,
                        preferred_element_type=jnp.float32)
    y_ref[...] = y[:, 0:1].reshape(BLK)


def _tc_y(emb, w2c):
    return pl.pallas_call(
        _tc_y_body,
        grid=(B // BLK,),
        in_specs=[
            pl.BlockSpec((BLK, D), lambda i: (i, 0)),
            pl.BlockSpec((D, 128), lambda i: (0, 0)),
        ],
        out_specs=pl.BlockSpec((BLK,), lambda i: (i,)),
        out_shape=jax.ShapeDtypeStruct((B,), jnp.float32),
    )(emb, w2c)


def _tc_add_body(y_ref, g_ref, b_ref, o_ref):
    o_ref[...] = (y_ref[...] + g_ref[...] + b_ref[0]).reshape(B, 1)


def _tc_add(y, g, b):
    return pl.pallas_call(
        _tc_add_body,
        in_specs=[
            pl.BlockSpec((B,), lambda: (0,)),
            pl.BlockSpec((B,), lambda: (0,)),
            pl.BlockSpec((1,), lambda: (0,)),
        ],
        out_specs=pl.BlockSpec((B, 1), lambda: (0, 0)),
        out_shape=jax.ShapeDtypeStruct((B, 1), jnp.float32),
    )(y, g, b)


def _sc_body(idx_hbm, s_hbm, g_hbm, idx_v, s_v, g_v):
    cid = lax.axis_index("c")
    sid = lax.axis_index("s")
    wid = cid * NS + sid
    base = wid * ROWS

    pltpu.sync_copy(idx_hbm.at[pl.ds(base, ROWS)], idx_v)
    pltpu.sync_copy(s_hbm, s_v)

    chunks = [s_v[pl.ds(c * L, L)] for c in range(VP // L)]
    dnums = lax.GatherDimensionNumbers(
        offset_dims=(), collapsed_slice_dims=(0,), start_index_map=(0,))

    def vreg_gather(vals, lo):
        return lax.gather(
            vals, lo[:, None], dnums, (1,),
            mode=lax.GatherScatterMode.PROMISE_IN_BOUNDS)

    for g in range(ROWS // L):
        gi = idx_v[pl.ds(g * L, L)]
        hi = gi >> 4
        lo = gi & 15
        sv = jnp.zeros((L,), jnp.float32)
        for c in range(VP // L):
            sv = jnp.where(hi == c, vreg_gather(chunks[c], lo), sv)
        g_v[pl.ds(g * L, L)] = sv

    pltpu.sync_copy(g_v, g_hbm.at[pl.ds(base, ROWS)])


@functools.partial(
    pl.kernel,
    mesh=plsc.VectorSubcoreMesh(core_axis_name="c", subcore_axis_name="s"),
    out_type=jax.ShapeDtypeStruct((B,), jnp.float32),
    scratch_types=[
        pltpu.VMEM((ROWS,), jnp.int32),
        pltpu.VMEM((VP,), jnp.float32),
        pltpu.VMEM((ROWS,), jnp.float32),
    ],
)
def _sc_gather(idx_hbm, s_hbm, g_hbm, *scratch):
    _sc_body(idx_hbm, s_hbm, g_hbm, *scratch)


def kernel(index, emb, pre_fix, W, b):
    w1 = W[:, :D]
    w2 = W[:, D:]
    zpad = jnp.zeros((D, 127), jnp.float32)
    w1c = jnp.concatenate([w1.T, zpad], axis=1)
    w2c = jnp.concatenate([w2.T, zpad], axis=1)
    pre_pad = jnp.pad(pre_fix, ((0, VP - V), (0, 0)))
    s = _tc_s(pre_pad, w1c)
    g = _sc_gather(index.astype(jnp.int32), s)
    y = _tc_y(emb, w2c)
    return _tc_add(y, g, b)


# speedup vs baseline: 1.2338x; 1.2338x over previous
import functools

import jax
import jax.numpy as jnp
from jax import lax
from jax.experimental import pallas as pl
from jax.experimental.pallas import tpu as pltpu
from jax.experimental.pallas import tpu_sc as plsc

L = 16
NC = 2
NS = 16
NW = NC * NS
B = 16384
D = 144
V = 180
VP = 192
ROWS = B // NW
BLK = 4096


_DOT_DIMS = (((1,), (0,)), ((), ()))


def _tc_body(emb_ref, pre_ref, w12_ref, b_ref, y_ref, s_ref):
    w12 = w12_ref[...]
    y = lax.dot_general(emb_ref[...], w12, _DOT_DIMS,
                        preferred_element_type=jnp.float32)
    y_ref[...] = y[:, 0:1].reshape(BLK) + b_ref[0]
    s = lax.dot_general(pre_ref[...], w12, _DOT_DIMS,
                        preferred_element_type=jnp.float32)
    s_ref[...] = s[:, 1:2].reshape(VP)


def _tc_dense(emb, pre_pad, w12, b):
    return pl.pallas_call(
        _tc_body,
        grid=(B // BLK,),
        in_specs=[
            pl.BlockSpec((BLK, D), lambda i: (i, 0)),
            pl.BlockSpec((VP, D), lambda i: (0, 0)),
            pl.BlockSpec((D, 128), lambda i: (0, 0)),
            pl.BlockSpec((1,), lambda i: (0,)),
        ],
        out_specs=[
            pl.BlockSpec((BLK,), lambda i: (i,)),
            pl.BlockSpec((VP,), lambda i: (0,)),
        ],
        out_shape=[
            jax.ShapeDtypeStruct((B,), jnp.float32),
            jax.ShapeDtypeStruct((VP,), jnp.float32),
        ],
    )(emb, pre_pad, w12, b)


def _sc_body(idx_hbm, y_hbm, s_hbm, out_hbm, idx_v, y_v, s_v, out_v):
    cid = lax.axis_index("c")
    sid = lax.axis_index("s")
    wid = cid * NS + sid
    base = wid * ROWS

    pltpu.sync_copy(idx_hbm.at[pl.ds(base, ROWS)], idx_v)
    pltpu.sync_copy(y_hbm.at[pl.ds(base, ROWS)], y_v)
    pltpu.sync_copy(s_hbm, s_v)

    chunks = [s_v[pl.ds(c * L, L)] for c in range(VP // L)]
    dnums = lax.GatherDimensionNumbers(
        offset_dims=(), collapsed_slice_dims=(0,), start_index_map=(0,))

    def vreg_gather(vals, lo):
        return lax.gather(
            vals, lo[:, None], dnums, (1,),
            mode=lax.GatherScatterMode.PROMISE_IN_BOUNDS)

    for g in range(ROWS // L):
        gi = idx_v[pl.ds(g * L, L)]
        hi = gi >> 4
        lo = gi & 15
        sv = jnp.zeros((L,), jnp.float32)
        for c in range(VP // L):
            sv = jnp.where(hi == c, vreg_gather(chunks[c], lo), sv)
        out_v[pl.ds(g * L, L)] = y_v[pl.ds(g * L, L)] + sv

    pltpu.sync_copy(out_v, out_hbm.at[pl.ds(base, ROWS)])


@functools.partial(
    pl.kernel,
    mesh=plsc.VectorSubcoreMesh(core_axis_name="c", subcore_axis_name="s"),
    out_type=jax.ShapeDtypeStruct((B,), jnp.float32),
    scratch_types=[
        pltpu.VMEM((ROWS,), jnp.int32),
        pltpu.VMEM((ROWS,), jnp.float32),
        pltpu.VMEM((VP,), jnp.float32),
        pltpu.VMEM((ROWS,), jnp.float32),
    ],
)
def _sc_gather_add(idx_hbm, y_hbm, s_hbm, out_hbm, *scratch):
    _sc_body(idx_hbm, y_hbm, s_hbm, out_hbm, *scratch)


def kernel(index, emb, pre_fix, W, b):
    w1 = W[:, :D]
    w2 = W[:, D:]
    w12 = jnp.concatenate(
        [w2.T, w1.T, jnp.zeros((D, 126), jnp.float32)], axis=1)
    pre_pad = jnp.pad(pre_fix, ((0, VP - V), (0, 0)))
    y, s = _tc_dense(emb, pre_pad, w12, b)
    out = _sc_gather_add(index.astype(jnp.int32), y, s)
    return out.reshape(B, 1)
